# baseline (speedup 1.0000x reference)
import jax
import jax.numpy as jnp
from jax.experimental import pallas as pl
from jax.experimental.pallas import tpu as pltpu

F32 = jnp.float32
BF16 = jnp.bfloat16


def _net_kernel(zt_ref, w1_ref, b1_ref, w2_ref, b2_ref, w3_ref, b3_ref,
                g_w_ref, g_b_ref,
                fw_hbm, ew1_hbm, ew2_hbm, ew3_hbm,
                fb_ref, eb1_ref, eb2_ref, eb3_ref,
                o_ref,
                y1, y2, y3c, fw_v, ew1_v, ew2_v, ew3_v,
                sem_f, sem_1, sem_2, sem_3):
    cp_f = pltpu.make_async_copy(fw_hbm, fw_v, sem_f)
    cp_1 = pltpu.make_async_copy(ew1_hbm, ew1_v, sem_1)
    cp_2 = pltpu.make_async_copy(ew2_hbm, ew2_v, sem_2)
    cp_3 = pltpu.make_async_copy(ew3_hbm, ew3_v, sem_3)
    cp_f.start()
    cp_1.start()
    cp_2.start()
    cp_3.start()

    w1 = w1_ref[...]
    b1 = b1_ref[...]
    for i in range(20):
        for j in range(0, 20, 2):
            pa = zt_ref[pl.ds(i, 2), pl.ds(j, 2), :, :].reshape(256, 128)
            pb = zt_ref[pl.ds(i, 2), pl.ds(j + 1, 2), :, :].reshape(256, 128)
            p = jnp.concatenate([pa, pb], axis=1)
            acc = jnp.dot(w1, p, preferred_element_type=F32)
            acc = jnp.maximum(acc + b1, 0.0)
            y1[i, j] = acc[:, :128]
            y1[i, j + 1] = acc[:, 128:]

    w2 = w2_ref[...]
    b2 = b2_ref[...]
    for i in range(9):
        for j in range(0, 8, 2):
            pa = y1[pl.ds(2 * i, 4), pl.ds(2 * j, 4), :, :].reshape(512, 128)
            pb = y1[pl.ds(2 * i, 4), pl.ds(2 * j + 2, 4), :, :].reshape(512, 128)
            p = jnp.concatenate([pa, pb], axis=1)
            acc = jnp.dot(w2, p, preferred_element_type=F32)
            acc = jnp.maximum(acc + b2, 0.0)
            y2[i, j] = acc[:, :128]
            y2[i, j + 1] = acc[:, 128:]
        p = y1[pl.ds(2 * i, 4), pl.ds(16, 4), :, :].reshape(512, 128)
        acc = jnp.dot(w2, p, preferred_element_type=F32)
        y2[i, 8] = jnp.maximum(acc + b2, 0.0)

    w3 = w3_ref[...]
    b3 = b3_ref[...]
    for i in range(7):
        for j in range(0, 6, 2):
            pa = y2[pl.ds(i, 3), pl.ds(j, 3), :, :].reshape(576, 128)
            pb = y2[pl.ds(i, 3), pl.ds(j + 1, 3), :, :].reshape(576, 128)
            p = jnp.concatenate([pa, pb], axis=1)
            acc = jnp.dot(w3, p, preferred_element_type=F32)
            acc = jnp.maximum(acc + b3, 0.0)
            y3c[:, i * 7 + j] = acc[:, :128]
            y3c[:, i * 7 + j + 1] = acc[:, 128:]
        p = y2[pl.ds(i, 3), pl.ds(6, 3), :, :].reshape(576, 128)
        acc = jnp.dot(w3, p, preferred_element_type=F32)
        y3c[:, i * 7 + 6] = jnp.maximum(acc + b3, 0.0)

    cp_f.wait()
    x = y3c[...].reshape(3136, 128)
    feats = jnp.maximum(
        jnp.dot(fw_v[...], x, preferred_element_type=F32) + fb_ref[...], 0.0)

    glog = jnp.dot(g_w_ref[...], feats, preferred_element_type=F32) + g_b_ref[...]
    row = jax.lax.broadcasted_iota(jnp.int32, (8, 128), 0)
    l = jnp.where(row < 6, glog, -1e30)
    m1 = jnp.max(l, axis=0, keepdims=True)
    i1 = jnp.min(jnp.where(l == m1, row, 127), axis=0, keepdims=True)
    l2 = jnp.where(row == i1, -1e30, l)
    m2 = jnp.max(l2, axis=0, keepdims=True)
    i2 = jnp.min(jnp.where(l2 == m2, row, 127), axis=0, keepdims=True)
    w1c = 1.0 / (1.0 + jnp.exp(m2 - m1))
    w2c = 1.0 - w1c

    cp_1.wait()
    cp_2.wait()
    cp_3.wait()
    acc = jnp.zeros((16, 128), F32)
    for e in range(6):
        h1 = jnp.maximum(
            jnp.dot(ew1_v[e], feats, preferred_element_type=F32) + eb1_ref[e],
            0.0)
        h2 = jnp.maximum(
            jnp.dot(ew2_v[e], h1, preferred_element_type=F32) + eb2_ref[e],
            0.0)
        oe = jnp.dot(ew3_v[e], h2, preferred_element_type=F32) + eb3_ref[e]
        coef = jnp.where(i1 == e, w1c, 0.0) + jnp.where(i2 == e, w2c, 0.0)
        acc = acc + coef * oe
    o_ref[...] = acc


def kernel(input, conv1_w, conv1_b, conv2_w, conv2_b, conv3_w, conv3_b,
           fc_w, fc_b, gate_w, gate_b, e_w1, e_b1, e_w2, e_b2, e_w3, e_b3):
    B = input.shape[0]

    t1 = input.reshape(B, 4, 21, 4, 21, 4).transpose(0, 2, 4, 1, 3, 5)
    zt = t1.reshape(B, 21 * 21 * 64).T.reshape(21, 21, 64, B)

    w1m = conv1_w.reshape(32, 4, 2, 4, 2, 4).transpose(0, 2, 4, 1, 3, 5)
    w1m = w1m.reshape(32, 256)
    w2m = conv2_w.transpose(0, 2, 3, 1).reshape(64, 512)
    w3m = conv3_w.transpose(0, 2, 3, 1).reshape(64, 576)
    gw8 = jnp.pad(gate_w, ((0, 2), (0, 0)))
    gb8 = jnp.pad(gate_b, (0, 2)).reshape(8, 1)

    E, Hd, D = e_w1.shape
    ew3p = jnp.pad(e_w3, ((0, 0), (0, 4), (0, 0)))
    eb3p = jnp.pad(e_b3, ((0, 0), (0, 4))).reshape(E, 16, 1)

    vmem = pltpu.VMEM
    final_t = pl.pallas_call(
        _net_kernel,
        in_specs=[pl.BlockSpec(memory_space=pltpu.VMEM)] * 9
        + [pl.BlockSpec(memory_space=pl.ANY)] * 4
        + [pl.BlockSpec(memory_space=pltpu.VMEM)] * 4,
        out_shape=jax.ShapeDtypeStruct((16, B), F32),
        scratch_shapes=[
            vmem((20, 20, 32, B), F32),
            vmem((9, 9, 64, B), F32),
            vmem((64, 49, B), F32),
            vmem((512, 3136), F32),
            vmem((E, Hd, D), F32),
            vmem((E, Hd, Hd), F32),
            vmem((E, 16, D), F32),
            pltpu.SemaphoreType.DMA,
            pltpu.SemaphoreType.DMA,
            pltpu.SemaphoreType.DMA,
            pltpu.SemaphoreType.DMA,
        ],
    )(zt, w1m, conv1_b.reshape(32, 1), w2m, conv2_b.reshape(64, 1),
      w3m, conv3_b.reshape(64, 1), gw8, gb8,
      fc_w, e_w1, e_w2, ew3p,
      fc_b.reshape(512, 1), e_b1.reshape(E, Hd, 1), e_b2.reshape(E, Hd, 1),
      eb3p)

    return final_t[:12].T
